# baseline (speedup 1.0000x reference)
import functools

import jax
import jax.numpy as jnp
from jax import lax
from jax.experimental import pallas as pl
from jax.experimental.pallas import tpu as pltpu

_K = 256
_G = 64


def _two_phase_kernel(jlo_ref, jhi_ref, x_ref, wt3_ref, g3_ref, bias_ref,
                      out_ref, xs_ref, *, B, G, K, FG, JB, NPH1):
    t = pl.program_id(0)

    @pl.when(t < NPH1)
    def _():
        blk = x_ref[...]
        blk3 = blk.reshape(B, JB, K).transpose(1, 0, 2)
        xs_ref[pl.ds(t * JB, JB)] = blk3

    @pl.when(t >= NPH1)
    def _():
        t2 = t - NPH1
        base = t2 * G

        gcol = lax.broadcasted_iota(jnp.int32, (FG, K), 0) // 4

        def body(j, acc):
            xk = xs_ref[j]
            wkT = wt3_ref[j]
            gk = g3_ref[pl.ds(j, 1), :]
            mask = (gk - base) == gcol
            wsel = jnp.broadcast_to(wkT[None, :, :], (G, 4, K)).reshape(FG, K)
            ft = jnp.where(mask, wsel, 0.0)
            return acc + lax.dot_general(
                xk, ft, (((1,), (1,)), ((), ())),
                preferred_element_type=jnp.float32)

        acc = lax.fori_loop(jlo_ref[t2], jhi_ref[t2], body,
                            jnp.zeros((B, FG), jnp.float32))
        out_ref[...] = jnp.tanh(acc + bias_ref[0])


def kernel(x, snp_gene, W, bias):
    B, NS = x.shape
    NG, L = bias.shape
    K, G = _K, _G
    FG = L * G

    NSUB = (NS + K - 1) // K
    NT = (NG + G - 1) // G
    NGP = NT * G

    sg = snp_gene.astype(jnp.int32)
    pad = NSUB * K - NS
    if pad:
        x = jnp.pad(x, ((0, 0), (0, pad)))
        sg = jnp.pad(sg, (0, pad), constant_values=NGP)
        W = jnp.pad(W, ((0, pad), (0, 0)))

    JB = 1
    for cand in (25, 16, 8, 5, 4, 2):
        if NSUB % cand == 0:
            JB = cand
            break
    NPH1 = NSUB // JB

    wt3 = W.T.reshape(L, NSUB, K).transpose(1, 0, 2)
    g3 = sg.reshape(NSUB, K)
    biasp = jnp.pad(bias, ((0, NGP - NG), (0, 0))).reshape(NT, 1, FG)

    bnd = (jnp.arange(NT + 1, dtype=jnp.int32) * G).astype(sg.dtype)
    starts = jnp.searchsorted(sg, bnd).astype(jnp.int32)
    jlo = starts[:-1] // K
    jhi = (starts[1:] + K - 1) // K

    grid_spec = pltpu.PrefetchScalarGridSpec(
        num_scalar_prefetch=2,
        grid=(NPH1 + NT,),
        in_specs=[
            pl.BlockSpec((B, JB * K),
                         lambda t, lo, hi: (0, jnp.minimum(t, NPH1 - 1))),
            pl.BlockSpec(wt3.shape, lambda t, lo, hi: (0, 0, 0)),
            pl.BlockSpec(g3.shape, lambda t, lo, hi: (0, 0)),
            pl.BlockSpec((1, 1, FG),
                         lambda t, lo, hi: (jnp.clip(t - NPH1, 0, NT - 1),
                                            0, 0)),
        ],
        out_specs=pl.BlockSpec(
            (B, FG),
            lambda t, lo, hi: (0, jnp.clip(t - NPH1, 0, NT - 1))),
        scratch_shapes=[pltpu.VMEM((NSUB, B, K), jnp.float32)],
    )
    return pl.pallas_call(
        functools.partial(_two_phase_kernel, B=B, G=G, K=K, FG=FG, JB=JB,
                          NPH1=NPH1),
        grid_spec=grid_spec,
        out_shape=jax.ShapeDtypeStruct((B, NG * L), jnp.float32),
    )(jlo, jhi, x, wt3, g3, biasp)


# speedup vs baseline: 1.0784x; 1.0784x over previous
import functools

import jax
import jax.numpy as jnp
from jax import lax
from jax.experimental import pallas as pl
from jax.experimental.pallas import tpu as pltpu

_K = 256
_G = 32


def _tile_kernel(jlo_ref, jhi_ref, x3_ref, wt3_ref, g3_ref, bias_ref, out_ref,
                 *, B, G, K, FG):
    t = pl.program_id(0)
    base = t * G

    gcol = lax.broadcasted_iota(jnp.int32, (FG, K), 0) // 4

    def body(j, acc):
        xk = x3_ref[j]
        wkT = wt3_ref[j]
        gk = g3_ref[pl.ds(j, 1), :]
        mask = (gk - base) == gcol
        wsel = jnp.broadcast_to(wkT[None, :, :], (G, 4, K)).reshape(FG, K)
        ft = jnp.where(mask, wsel, 0.0)
        return acc + lax.dot_general(
            xk, ft, (((1,), (1,)), ((), ())),
            preferred_element_type=jnp.float32)

    acc = lax.fori_loop(jlo_ref[t], jhi_ref[t], body,
                        jnp.zeros((B, FG), jnp.float32))
    out_ref[...] = jnp.tanh(acc + bias_ref[0])


def kernel(x, snp_gene, W, bias):
    B, NS = x.shape
    NG, L = bias.shape
    K, G = _K, _G
    FG = L * G

    NSUB = (NS + K - 1) // K
    NT = (NG + G - 1) // G
    NGP = NT * G

    sg = snp_gene.astype(jnp.int32)
    pad = NSUB * K - NS
    if pad:
        x = jnp.pad(x, ((0, 0), (0, pad)))
        sg = jnp.pad(sg, (0, pad), constant_values=NGP)
        W = jnp.pad(W, ((0, pad), (0, 0)))

    x3 = x.reshape(B, NSUB, K).transpose(1, 0, 2)
    wt3 = W.T.reshape(L, NSUB, K).transpose(1, 0, 2)
    g3 = sg.reshape(NSUB, K)
    biasp = jnp.pad(bias, ((0, NGP - NG), (0, 0))).reshape(NT, 1, FG)

    bnd = (jnp.arange(NT + 1, dtype=jnp.int32) * G).astype(sg.dtype)
    starts = jnp.searchsorted(sg, bnd).astype(jnp.int32)
    jlo = starts[:-1] // K
    jhi = (starts[1:] + K - 1) // K

    grid_spec = pltpu.PrefetchScalarGridSpec(
        num_scalar_prefetch=2,
        grid=(NT,),
        in_specs=[
            pl.BlockSpec(x3.shape, lambda t, lo, hi: (0, 0, 0)),
            pl.BlockSpec(wt3.shape, lambda t, lo, hi: (0, 0, 0)),
            pl.BlockSpec(g3.shape, lambda t, lo, hi: (0, 0)),
            pl.BlockSpec((1, 1, FG), lambda t, lo, hi: (t, 0, 0)),
        ],
        out_specs=pl.BlockSpec((B, FG), lambda t, lo, hi: (0, t)),
    )
    return pl.pallas_call(
        functools.partial(_tile_kernel, B=B, G=G, K=K, FG=FG),
        grid_spec=grid_spec,
        out_shape=jax.ShapeDtypeStruct((B, NG * L), jnp.float32),
    )(jlo, jhi, x3, wt3, g3, biasp)


# speedup vs baseline: 1.2709x; 1.1785x over previous
import functools

import jax
import jax.numpy as jnp
from jax import lax
from jax.experimental import pallas as pl
from jax.experimental.pallas import tpu as pltpu

_K = 256
_G = 128


def _tile_kernel(jlo_ref, jhi_ref, x3_ref, wt3_ref, g3_ref, bias_ref, out_ref,
                 *, B, G, K, FG):
    t = pl.program_id(0)
    base = t * G

    gcol = lax.broadcasted_iota(jnp.int32, (FG, K), 0) // 4

    def body(j, acc):
        xk = x3_ref[j]
        wkT = wt3_ref[j]
        gk = g3_ref[pl.ds(j, 1), :]
        mask = (gk - base) == gcol
        wsel = jnp.broadcast_to(wkT[None, :, :], (G, 4, K)).reshape(FG, K)
        ft = jnp.where(mask, wsel, 0.0)
        return acc + lax.dot_general(
            xk, ft, (((1,), (1,)), ((), ())),
            preferred_element_type=jnp.float32)

    acc = lax.fori_loop(jlo_ref[t], jhi_ref[t], body,
                        jnp.zeros((B, FG), jnp.float32))
    out_ref[...] = jnp.tanh(acc + bias_ref[0])


def kernel(x, snp_gene, W, bias):
    B, NS = x.shape
    NG, L = bias.shape
    K, G = _K, _G
    FG = L * G

    NSUB = (NS + K - 1) // K
    NT = (NG + G - 1) // G
    NGP = NT * G

    sg = snp_gene.astype(jnp.int32)
    pad = NSUB * K - NS
    if pad:
        x = jnp.pad(x, ((0, 0), (0, pad)))
        sg = jnp.pad(sg, (0, pad), constant_values=NGP)
        W = jnp.pad(W, ((0, pad), (0, 0)))

    x3 = x.reshape(B, NSUB, K).transpose(1, 0, 2)
    wt3 = W.T.reshape(L, NSUB, K).transpose(1, 0, 2)
    g3 = sg.reshape(NSUB, K)
    biasp = jnp.pad(bias, ((0, NGP - NG), (0, 0))).reshape(NT, 1, FG)

    bnd = (jnp.arange(NT + 1, dtype=jnp.int32) * G).astype(sg.dtype)
    starts = jnp.searchsorted(sg, bnd).astype(jnp.int32)
    jlo = starts[:-1] // K
    jhi = (starts[1:] + K - 1) // K

    grid_spec = pltpu.PrefetchScalarGridSpec(
        num_scalar_prefetch=2,
        grid=(NT,),
        in_specs=[
            pl.BlockSpec(x3.shape, lambda t, lo, hi: (0, 0, 0)),
            pl.BlockSpec(wt3.shape, lambda t, lo, hi: (0, 0, 0)),
            pl.BlockSpec(g3.shape, lambda t, lo, hi: (0, 0)),
            pl.BlockSpec((1, 1, FG), lambda t, lo, hi: (t, 0, 0)),
        ],
        out_specs=pl.BlockSpec((B, FG), lambda t, lo, hi: (0, t)),
    )
    return pl.pallas_call(
        functools.partial(_tile_kernel, B=B, G=G, K=K, FG=FG),
        grid_spec=grid_spec,
        out_shape=jax.ShapeDtypeStruct((B, NG * L), jnp.float32),
    )(jlo, jhi, x3, wt3, g3, biasp)
